# Initial kernel scaffold; baseline (speedup 1.0000x reference)
import jax
import jax.numpy as jnp
from jax.experimental import pallas as pl


def kernel(node_repr, edge_repr, node_neighbor, edge_neighbor, W_deg, W_self, bias):
    raise NotImplementedError("write your pallas kernel here")


# speedup vs baseline: 2.7510x; 2.7510x over previous
import functools

import jax
import jax.numpy as jnp
from jax import lax
from jax.experimental import pallas as pl
from jax.experimental.pallas import tpu as pltpu
from jax.experimental.pallas import tpu_sc as plsc

N_NODES = 10000
N_EDGES = 320000
DEGREE = 32
NODE_SIZE = 128
EDGE_SIZE = 16
OUT_SIZE = 128
EPS = 1e-5

NUM_WORKERS = 32
CHUNK = 4
IDX_PER_CHUNK = CHUNK * DEGREE
NUM_CHUNKS = N_NODES // CHUNK
CHUNKS_PER_WORKER = -(-NUM_CHUNKS // NUM_WORKERS)
LANES = 16
NVEC = NODE_SIZE // LANES


def _sc_body(node_hbm, edge_hbm, nn_hbm, en_hbm, nsum_hbm, esum_hbm,
             nidx_v, eidx_v, nrows_v, erows_v, nout_v, eout_v, sem_n, sem_e):
    cid = lax.axis_index("c")
    sid = lax.axis_index("s")
    wid = sid * 2 + cid

    def chunk_body(k, carry):
        chunk = wid + k * NUM_WORKERS

        @pl.when(chunk < NUM_CHUNKS)
        def _():
            base = chunk * IDX_PER_CHUNK
            pltpu.sync_copy(nn_hbm.at[pl.ds(base, IDX_PER_CHUNK)], nidx_v)
            pltpu.sync_copy(en_hbm.at[pl.ds(base, IDX_PER_CHUNK)], eidx_v)
            cp_n = pltpu.async_copy(node_hbm.at[nidx_v], nrows_v, sem_n)
            cp_e = pltpu.async_copy(edge_hbm.at[eidx_v], erows_v, sem_e)
            cp_n.wait()
            cp_e.wait()

            for n in range(CHUNK):
                def red(j, acc):
                    row = n * DEGREE + j
                    new_n = tuple(
                        acc[v] + nrows_v[row, pl.ds(v * LANES, LANES)]
                        for v in range(NVEC)
                    )
                    return new_n + (acc[NVEC] + erows_v[row, :],)

                zero = jnp.zeros((LANES,), jnp.float32)
                acc = lax.fori_loop(0, DEGREE, red, (zero,) * (NVEC + 1))
                for v in range(NVEC):
                    nout_v[n, pl.ds(v * LANES, LANES)] = acc[v]
                eout_v[n, :] = acc[NVEC]

            pltpu.sync_copy(nout_v, nsum_hbm.at[pl.ds(chunk * CHUNK, CHUNK)])
            pltpu.sync_copy(eout_v, esum_hbm.at[pl.ds(chunk * CHUNK, CHUNK)])

        return carry

    lax.fori_loop(0, CHUNKS_PER_WORKER, chunk_body, 0)


@functools.partial(jax.jit, static_argnums=())
def _sc_gather_sum(node_repr, edge_repr, nn_flat, en_flat):
    mesh = plsc.VectorSubcoreMesh(core_axis_name="c", subcore_axis_name="s")
    kern = pl.kernel(
        _sc_body,
        mesh=mesh,
        compiler_params=pltpu.CompilerParams(use_tc_tiling_on_sc=False),
        out_type=[
            jax.ShapeDtypeStruct((N_NODES, NODE_SIZE), jnp.float32),
            jax.ShapeDtypeStruct((N_NODES, EDGE_SIZE), jnp.float32),
        ],
        scratch_types=[
            pltpu.VMEM((IDX_PER_CHUNK,), jnp.int32),
            pltpu.VMEM((IDX_PER_CHUNK,), jnp.int32),
            pltpu.VMEM((IDX_PER_CHUNK, NODE_SIZE), jnp.float32),
            pltpu.VMEM((IDX_PER_CHUNK, EDGE_SIZE), jnp.float32),
            pltpu.VMEM((CHUNK, NODE_SIZE), jnp.float32),
            pltpu.VMEM((CHUNK, EDGE_SIZE), jnp.float32),
            pltpu.SemaphoreType.DMA,
            pltpu.SemaphoreType.DMA,
        ],
    )
    return kern(node_repr, edge_repr, nn_flat, en_flat)


def _tc_body(nsum_ref, esum_ref, node_ref, wdn_ref, wde_ref, ws_ref, bias_ref,
             out_ref):
    act = jnp.dot(nsum_ref[:], wdn_ref[:], preferred_element_type=jnp.float32)
    act = act + jnp.dot(esum_ref[:], wde_ref[:],
                        preferred_element_type=jnp.float32)
    act = act + jnp.dot(node_ref[:], ws_ref[:],
                        preferred_element_type=jnp.float32)
    act = act + bias_ref[:]
    mean = jnp.mean(act, axis=0, keepdims=True)
    cent = act - mean
    var = jnp.mean(cent * cent, axis=0, keepdims=True)
    out_ref[:] = jnp.maximum(cent * lax.rsqrt(var + EPS), 0.0)


def _tc_finish(nsum, esum, node_repr, wdn, wde, ws, bias):
    return pl.pallas_call(
        _tc_body,
        out_shape=jax.ShapeDtypeStruct((N_NODES, OUT_SIZE), jnp.float32),
    )(nsum, esum, node_repr, wdn, wde, ws, bias)


def kernel(node_repr, edge_repr, node_neighbor, edge_neighbor, W_deg, W_self,
           bias):
    nn_flat = node_neighbor.reshape(-1)
    en_flat = edge_neighbor.reshape(-1)
    nsum, esum = _sc_gather_sum(node_repr, edge_repr, nn_flat, en_flat)
    return _tc_finish(nsum, esum, node_repr, W_deg[:NODE_SIZE],
                      W_deg[NODE_SIZE:], W_self, bias)


# speedup vs baseline: 4.4588x; 1.6208x over previous
import functools

import jax
import jax.numpy as jnp
from jax import lax
from jax.experimental import pallas as pl
from jax.experimental.pallas import tpu as pltpu
from jax.experimental.pallas import tpu_sc as plsc

N_NODES = 10000
N_EDGES = 320000
DEGREE = 32
NODE_SIZE = 128
EDGE_SIZE = 16
OUT_SIZE = 128
EPS = 1e-5

NUM_WORKERS = 32
CHUNK = 4
IDX_PER_CHUNK = CHUNK * DEGREE
NUM_CHUNKS = N_NODES // CHUNK
MAX_CPW = -(-NUM_CHUNKS // NUM_WORKERS)
MAX_NPW = MAX_CPW * CHUNK
LANES = 16
NVEC = NODE_SIZE // LANES
NBUF = 2


def _sc_body(node_hbm, edge_hbm, nn_hbm, en_hbm, nsum_hbm, esum_hbm,
             nidx_v, eidx_v, nrows, erows, nout_v, eout_v, sems_n, sems_e):
    cid = lax.axis_index("c")
    sid = lax.axis_index("s")
    wid = sid * 2 + cid
    base = wid * NUM_CHUNKS // NUM_WORKERS
    cnt = (wid + 1) * NUM_CHUNKS // NUM_WORKERS - base

    pltpu.sync_copy(nn_hbm.at[pl.ds(base, MAX_CPW)], nidx_v)
    pltpu.sync_copy(en_hbm.at[pl.ds(base, MAX_CPW)], eidx_v)

    def issue(k, b):
        pltpu.async_copy(node_hbm.at[nidx_v.at[k]], nrows[b], sems_n[b])
        pltpu.async_copy(edge_hbm.at[eidx_v.at[k]], erows[b], sems_e[b])

    def drain(k, b):
        pltpu.make_async_copy(node_hbm.at[nidx_v.at[k]], nrows[b],
                              sems_n[b]).wait()
        pltpu.make_async_copy(edge_hbm.at[eidx_v.at[k]], erows[b],
                              sems_e[b]).wait()

    def reduce_chunk(k, b):
        for n in range(CHUNK):
            def red(j, acc):
                new = acc
                for jj in range(4):
                    row = n * DEGREE + j * 4 + jj
                    new = tuple(
                        new[v] + nrows[b][row, pl.ds(v * LANES, LANES)]
                        for v in range(NVEC)
                    ) + (new[NVEC] + erows[b][row, :],)
                return new

            zero = jnp.zeros((LANES,), jnp.float32)
            acc = lax.fori_loop(0, DEGREE // 4, red, (zero,) * (NVEC + 1))
            out_row = k * CHUNK + n
            for v in range(NVEC):
                nout_v[out_row, pl.ds(v * LANES, LANES)] = acc[v]
            eout_v[out_row, :] = acc[NVEC]

    for b in range(NBUF):
        issue(b, b)

    def pair_body(i, carry):
        for b in range(NBUF):
            k = i * NBUF + b

            @pl.when(k < cnt)
            def _():
                drain(k, b)
                reduce_chunk(k, b)

                @pl.when(k + NBUF < cnt)
                def _():
                    issue(k + NBUF, b)

        return carry

    lax.fori_loop(0, -(-MAX_CPW // NBUF), pair_body, 0)

    row0 = base * CHUNK

    @pl.when(cnt == MAX_CPW)
    def _():
        pltpu.sync_copy(nout_v, nsum_hbm.at[pl.ds(row0, MAX_NPW)])
        pltpu.sync_copy(eout_v, esum_hbm.at[pl.ds(row0, MAX_NPW)])

    @pl.when(cnt == MAX_CPW - 1)
    def _():
        nrows_small = MAX_NPW - CHUNK
        pltpu.sync_copy(nout_v.at[pl.ds(0, nrows_small)],
                        nsum_hbm.at[pl.ds(row0, nrows_small)])
        pltpu.sync_copy(eout_v.at[pl.ds(0, nrows_small)],
                        esum_hbm.at[pl.ds(row0, nrows_small)])


@functools.partial(jax.jit, static_argnums=())
def _sc_gather_sum(node_repr, edge_repr, nn2d, en2d):
    mesh = plsc.VectorSubcoreMesh(core_axis_name="c", subcore_axis_name="s")
    kern = pl.kernel(
        _sc_body,
        mesh=mesh,
        compiler_params=pltpu.CompilerParams(use_tc_tiling_on_sc=False),
        out_type=[
            jax.ShapeDtypeStruct((N_NODES, NODE_SIZE), jnp.float32),
            jax.ShapeDtypeStruct((N_NODES, EDGE_SIZE), jnp.float32),
        ],
        scratch_types=[
            pltpu.VMEM((MAX_CPW, IDX_PER_CHUNK), jnp.int32),
            pltpu.VMEM((MAX_CPW, IDX_PER_CHUNK), jnp.int32),
            [pltpu.VMEM((IDX_PER_CHUNK, NODE_SIZE), jnp.float32)
             for _ in range(NBUF)],
            [pltpu.VMEM((IDX_PER_CHUNK, EDGE_SIZE), jnp.float32)
             for _ in range(NBUF)],
            pltpu.VMEM((MAX_NPW, NODE_SIZE), jnp.float32),
            pltpu.VMEM((MAX_NPW, EDGE_SIZE), jnp.float32),
            [pltpu.SemaphoreType.DMA for _ in range(NBUF)],
            [pltpu.SemaphoreType.DMA for _ in range(NBUF)],
        ],
    )
    return kern(node_repr, edge_repr, nn2d, en2d)


def _tc_body(nsum_ref, esum_ref, node_ref, wdn_ref, wde_ref, ws_ref, bias_ref,
             out_ref):
    act = jnp.dot(nsum_ref[:], wdn_ref[:], preferred_element_type=jnp.float32)
    act = act + jnp.dot(esum_ref[:], wde_ref[:],
                        preferred_element_type=jnp.float32)
    act = act + jnp.dot(node_ref[:], ws_ref[:],
                        preferred_element_type=jnp.float32)
    act = act + bias_ref[:]
    mean = jnp.mean(act, axis=0, keepdims=True)
    cent = act - mean
    var = jnp.mean(cent * cent, axis=0, keepdims=True)
    out_ref[:] = jnp.maximum(cent * lax.rsqrt(var + EPS), 0.0)


def _tc_finish(nsum, esum, node_repr, wdn, wde, ws, bias):
    return pl.pallas_call(
        _tc_body,
        out_shape=jax.ShapeDtypeStruct((N_NODES, OUT_SIZE), jnp.float32),
    )(nsum, esum, node_repr, wdn, wde, ws, bias)


def kernel(node_repr, edge_repr, node_neighbor, edge_neighbor, W_deg, W_self,
           bias):
    nn2d = node_neighbor.reshape(NUM_CHUNKS, IDX_PER_CHUNK)
    en2d = edge_neighbor.reshape(NUM_CHUNKS, IDX_PER_CHUNK)
    nsum, esum = _sc_gather_sum(node_repr, edge_repr, nn2d, en2d)
    return _tc_finish(nsum, esum, node_repr, W_deg[:NODE_SIZE],
                      W_deg[NODE_SIZE:], W_self, bias)
